# baseline (speedup 1.0000x reference)
import functools

import jax
import jax.numpy as jnp
import numpy as np
from jax.experimental import pallas as pl
from jax.experimental.pallas import tpu as pltpu

_ANCHORS = np.array([[116.0, 90.0], [156.0, 198.0], [373.0, 326.0]], dtype=np.float32)
_NUM_CLASSES = 80
_IMG_DIM = 608.0
_NBUF = 3


def _decode_kernel(x_hbm, o_hbm, ibuf, obuf, isem, osem, *, G, stride, anchors, nsteps):
    nA = anchors.shape[0]
    GG = G * G
    nch = _NUM_CLASSES + 5
    i = pl.program_id(0)
    slot = jax.lax.rem(i, _NBUF)
    b = jax.lax.div(i, nA)
    a = jax.lax.rem(i, nA)

    H = 48
    R = GG // 2

    def in_copy(bb, aa, sl):
        pltpu.make_async_copy(
            x_hbm.at[bb, pl.ds(aa * nch, H)], ibuf.at[sl, pl.ds(0, H)], isem.at[sl, 0]
        ).start()
        pltpu.make_async_copy(
            x_hbm.at[bb, pl.ds(aa * nch + H, nch - H)],
            ibuf.at[sl, pl.ds(H, nch - H)],
            isem.at[sl, 1],
        ).start()

    @pl.when(i == 0)
    def _warmup():
        for k in range(_NBUF):
            in_copy(k // nA, k % nA, k)

    pltpu.make_async_copy(
        x_hbm.at[0, pl.ds(0, H)], ibuf.at[slot, pl.ds(0, H)], isem.at[slot, 0]
    ).wait()
    pltpu.make_async_copy(
        x_hbm.at[0, pl.ds(0, nch - H)],
        ibuf.at[slot, pl.ds(H, nch - H)],
        isem.at[slot, 1],
    ).wait()

    @pl.when(i >= _NBUF)
    def _wait_out():
        for h in range(2):
            pltpu.make_async_copy(
                obuf.at[slot, pl.ds(h * R, R)],
                o_hbm.at[0, pl.ds(0, R), :],
                osem.at[slot, h],
            ).wait()

    X = ibuf[slot]

    sig = jax.nn.sigmoid(X)

    top = X[0:8]
    row8 = jax.lax.broadcasted_iota(jnp.int32, (8, G, G), 0)
    gy = jax.lax.broadcasted_iota(jnp.int32, (1, G, G), 1).astype(jnp.float32)
    gx = jax.lax.broadcasted_iota(jnp.int32, (1, G, G), 2).astype(jnp.float32)

    ex = jnp.exp(top)
    sig8 = sig[0:8]

    aw = jnp.where(a == 0, anchors[0, 0], jnp.where(a == 1, anchors[1, 0], anchors[2, 0]))
    ah = jnp.where(a == 0, anchors[0, 1], jnp.where(a == 1, anchors[1, 1], anchors[2, 1]))

    base = jnp.where((row8 == 2) | (row8 == 3), ex, sig8)
    add = jnp.where(row8 == 0, gx, jnp.where(row8 == 1, gy, 0.0))
    scale = jnp.where(
        row8 < 2, stride, jnp.where(row8 == 2, aw, jnp.where(row8 == 3, ah, 1.0))
    )
    top_out = (base + add) * scale

    y = jnp.concatenate([top_out, sig[8:]], axis=0)

    for gyi in range(G):
        obuf[slot, gyi * G : (gyi + 1) * G, :] = y[:, gyi, :].T

    for h in range(2):
        pltpu.make_async_copy(
            obuf.at[slot, pl.ds(h * R, R)],
            o_hbm.at[b, pl.ds(a * GG + h * R, R), :],
            osem.at[slot, h],
        ).start()

    @pl.when(i + _NBUF < nsteps)
    def _prefetch():
        bn = jax.lax.div(i + _NBUF, nA)
        an = jax.lax.rem(i + _NBUF, nA)
        in_copy(bn, an, slot)

    @pl.when(i == nsteps - 1)
    def _drain():
        for k in range(_NBUF):
            for h in range(2):
                pltpu.make_async_copy(
                    obuf.at[k, pl.ds(h * R, R)],
                    o_hbm.at[0, pl.ds(0, R), :],
                    osem.at[k, h],
                ).wait()


def kernel(x):
    B = x.shape[0]
    G = x.shape[2]
    nA = _ANCHORS.shape[0]
    nch = _NUM_CLASSES + 5
    GG = G * G
    stride = _IMG_DIM / G
    nsteps = B * nA

    out = pl.pallas_call(
        functools.partial(
            _decode_kernel, G=G, stride=stride, anchors=_ANCHORS, nsteps=nsteps
        ),
        grid=(nsteps,),
        in_specs=[pl.BlockSpec(memory_space=pltpu.MemorySpace.HBM)],
        out_specs=pl.BlockSpec(memory_space=pltpu.MemorySpace.HBM),
        out_shape=jax.ShapeDtypeStruct((B, nA * GG, nch), jnp.float32),
        scratch_shapes=[
            pltpu.VMEM((_NBUF, nch, G, G), jnp.float32),
            pltpu.VMEM((_NBUF, GG, nch), jnp.float32),
            pltpu.SemaphoreType.DMA((_NBUF, 2)),
            pltpu.SemaphoreType.DMA((_NBUF, 2)),
        ],
        compiler_params=pltpu.CompilerParams(
            dimension_semantics=("arbitrary",),
        ),
    )(x)

    return out


# speedup vs baseline: 1.0010x; 1.0010x over previous
import functools

import jax
import jax.numpy as jnp
import numpy as np
from jax.experimental import pallas as pl
from jax.experimental.pallas import tpu as pltpu

_ANCHORS = np.array([[116.0, 90.0], [156.0, 198.0], [373.0, 326.0]], dtype=np.float32)
_NUM_CLASSES = 80
_IMG_DIM = 608.0
_NBUF = 3


def _decode_kernel(x_hbm, o_hbm, ibuf, obuf, isem, osem, *, G, stride, anchors, nsteps):
    nA = anchors.shape[0]
    GG = G * G
    nch = _NUM_CLASSES + 5
    i = pl.program_id(0)
    slot = jax.lax.rem(i, _NBUF)
    b = jax.lax.div(i, nA)
    a = jax.lax.rem(i, nA)

    @pl.when(i == 0)
    def _warmup():
        for k in range(_NBUF):
            pltpu.make_async_copy(
                x_hbm.at[k // nA, pl.ds((k % nA) * nch, nch)], ibuf.at[k], isem.at[k]
            ).start()

    pltpu.make_async_copy(
        x_hbm.at[0, pl.ds(0, nch)], ibuf.at[slot], isem.at[slot]
    ).wait()

    @pl.when(i >= _NBUF)
    def _wait_out():
        pltpu.make_async_copy(
            obuf.at[slot], o_hbm.at[0, pl.ds(0, GG), :], osem.at[slot]
        ).wait()

    X = ibuf[slot]

    sig = jax.nn.sigmoid(X)

    top = X[0:8]
    row8 = jax.lax.broadcasted_iota(jnp.int32, (8, G, G), 0)
    gy = jax.lax.broadcasted_iota(jnp.int32, (1, G, G), 1).astype(jnp.float32)
    gx = jax.lax.broadcasted_iota(jnp.int32, (1, G, G), 2).astype(jnp.float32)

    ex = jnp.exp(top)
    sig8 = sig[0:8]

    aw = jnp.where(a == 0, anchors[0, 0], jnp.where(a == 1, anchors[1, 0], anchors[2, 0]))
    ah = jnp.where(a == 0, anchors[0, 1], jnp.where(a == 1, anchors[1, 1], anchors[2, 1]))

    base = jnp.where((row8 == 2) | (row8 == 3), ex, sig8)
    add = jnp.where(row8 == 0, gx, jnp.where(row8 == 1, gy, 0.0))
    scale = jnp.where(
        row8 < 2, stride, jnp.where(row8 == 2, aw, jnp.where(row8 == 3, ah, 1.0))
    )
    top_out = (base + add) * scale

    y = jnp.concatenate([top_out, sig[8:]], axis=0)

    for gyi in range(G):
        obuf[slot, gyi * G : (gyi + 1) * G, :] = y[:, gyi, :].T

    pltpu.make_async_copy(
        obuf.at[slot], o_hbm.at[b, pl.ds(a * GG, GG), :], osem.at[slot]
    ).start()

    @pl.when(i + _NBUF < nsteps)
    def _prefetch():
        bn = jax.lax.div(i + _NBUF, nA)
        an = jax.lax.rem(i + _NBUF, nA)
        pltpu.make_async_copy(
            x_hbm.at[bn, pl.ds(an * nch, nch)], ibuf.at[slot], isem.at[slot]
        ).start()

    @pl.when(i == nsteps - 1)
    def _drain():
        for k in range(_NBUF):
            pltpu.make_async_copy(
                obuf.at[k], o_hbm.at[0, pl.ds(0, GG), :], osem.at[k]
            ).wait()


def kernel(x):
    B = x.shape[0]
    G = x.shape[2]
    nA = _ANCHORS.shape[0]
    nch = _NUM_CLASSES + 5
    GG = G * G
    stride = _IMG_DIM / G
    nsteps = B * nA

    out = pl.pallas_call(
        functools.partial(
            _decode_kernel, G=G, stride=stride, anchors=_ANCHORS, nsteps=nsteps
        ),
        grid=(nsteps,),
        in_specs=[pl.BlockSpec(memory_space=pltpu.MemorySpace.HBM)],
        out_specs=pl.BlockSpec(memory_space=pltpu.MemorySpace.HBM),
        out_shape=jax.ShapeDtypeStruct((B, nA * GG, nch), jnp.float32),
        scratch_shapes=[
            pltpu.VMEM((_NBUF, nch, G, G), jnp.float32),
            pltpu.VMEM((_NBUF, GG, nch), jnp.float32),
            pltpu.SemaphoreType.DMA((_NBUF,)),
            pltpu.SemaphoreType.DMA((_NBUF,)),
        ],
        compiler_params=pltpu.CompilerParams(
            dimension_semantics=("arbitrary",),
        ),
    )(x)

    return out
